# baseline (speedup 1.0000x reference)
import numpy as np
import jax
import jax.numpy as jnp
from jax.experimental import pallas as pl
from jax.experimental.pallas import tpu as pltpu

_ALPHA = 4
_CHUNK = 8


def kernel(frames):
    C, T, H, W = frames.shape
    HW = H * W
    Ts = T // _ALPHA
    idx = np.linspace(0, T - 1, Ts).astype(np.int32)
    nj = T // _CHUNK

    fpt = HW // (8 * 128)
    ntiles = C * T * fpt
    f3 = frames.reshape(ntiles, 8, 128)
    cpt = _CHUNK * fpt

    def body(in_ref, slow_ref, fast_ref, buf, sin, sout):
        ins = []
        n = 0
        for c in range(C):
            for j in range(nj):
                sl = pl.ds((c * nj + j) * cpt, cpt)
                cp = pltpu.make_async_copy(in_ref.at[sl], buf.at[sl], sin.at[n])
                cp.start()
                ins.append((c, j, cp))
                n += 1
        outs = []
        for c, j, cp in ins:
            cp.wait()
            sl = pl.ds((c * nj + j) * cpt, cpt)
            o = pltpu.make_async_copy(buf.at[sl], fast_ref.at[sl], sout)
            o.start()
            outs.append(o)
            lo, hi = j * _CHUNK, (j + 1) * _CHUNK
            for p, g in enumerate(idx):
                if lo <= g < hi:
                    ssrc = pl.ds((c * T + int(g)) * fpt, fpt)
                    sdst = pl.ds((c * Ts + p) * fpt, fpt)
                    o2 = pltpu.make_async_copy(
                        buf.at[ssrc], slow_ref.at[sdst], sout)
                    o2.start()
                    outs.append(o2)
        for o in outs:
            o.wait()

    slow3, fast3 = pl.pallas_call(
        body,
        in_specs=[pl.BlockSpec(memory_space=pl.ANY)],
        out_specs=[
            pl.BlockSpec(memory_space=pl.ANY),
            pl.BlockSpec(memory_space=pl.ANY),
        ],
        out_shape=[
            jax.ShapeDtypeStruct((C * Ts * fpt, 8, 128), frames.dtype),
            jax.ShapeDtypeStruct((ntiles, 8, 128), frames.dtype),
        ],
        scratch_shapes=[
            pltpu.VMEM((ntiles, 8, 128), frames.dtype),
            pltpu.SemaphoreType.DMA((C * nj,)),
            pltpu.SemaphoreType.DMA,
        ],
    )(f3)

    return (slow3.reshape(C, Ts, H, W), fast3.reshape(C, T, H, W))


# speedup vs baseline: 1.2694x; 1.2694x over previous
import numpy as np
import jax
import jax.numpy as jnp
from jax.experimental import pallas as pl
from jax.experimental.pallas import tpu as pltpu

_ALPHA = 4


def kernel(frames):
    C, T, H, W = frames.shape
    HW = H * W
    Ts = T // _ALPHA
    idx = np.linspace(0, T - 1, Ts).astype(np.int32)

    fpt = HW // (8 * 128)
    f3 = frames.reshape(C * T * fpt, 8, 128)

    def body(in_ref, slow_ref, buf, sin, sout):
        ins = []
        n = 0
        for c in range(C):
            for p, g in enumerate(idx):
                src = pl.ds((c * T + int(g)) * fpt, fpt)
                dst = pl.ds((c * Ts + p) * fpt, fpt)
                cp = pltpu.make_async_copy(in_ref.at[src], buf.at[dst], sin.at[n])
                cp.start()
                ins.append((dst, cp))
                n += 1
        outs = []
        for dst, cp in ins:
            cp.wait()
            o = pltpu.make_async_copy(buf.at[dst], slow_ref.at[dst], sout)
            o.start()
            outs.append(o)
        for o in outs:
            o.wait()

    slow3 = pl.pallas_call(
        body,
        in_specs=[pl.BlockSpec(memory_space=pl.ANY)],
        out_specs=pl.BlockSpec(memory_space=pl.ANY),
        out_shape=jax.ShapeDtypeStruct((C * Ts * fpt, 8, 128), frames.dtype),
        scratch_shapes=[
            pltpu.VMEM((C * Ts * fpt, 8, 128), frames.dtype),
            pltpu.SemaphoreType.DMA((C * Ts,)),
            pltpu.SemaphoreType.DMA,
        ],
    )(f3)

    return (slow3.reshape(C, Ts, H, W), frames)


# speedup vs baseline: 3.4339x; 2.7052x over previous
import numpy as np
import jax
import jax.numpy as jnp
from jax.experimental import pallas as pl
from jax.experimental.pallas import tpu as pltpu

_ALPHA = 4


def kernel(frames):
    C, T, H, W = frames.shape
    Ts = T // _ALPHA
    idx = np.linspace(0, T - 1, Ts).astype(np.int32)

    def body(in_ref, slow_ref, buf, sin, sout):
        ins = []
        n = 0
        for c in range(C):
            for p, g in enumerate(idx):
                cp = pltpu.make_async_copy(
                    in_ref.at[c, int(g)], buf.at[n], sin.at[n])
                cp.start()
                ins.append((c, p, cp))
                n += 1
        outs = []
        n = 0
        for c, p, cp in ins:
            cp.wait()
            o = pltpu.make_async_copy(buf.at[n], slow_ref.at[c, p], sout)
            o.start()
            outs.append(o)
            n += 1
        for o in outs:
            o.wait()

    slow = pl.pallas_call(
        body,
        in_specs=[pl.BlockSpec(memory_space=pl.ANY)],
        out_specs=pl.BlockSpec(memory_space=pl.ANY),
        out_shape=jax.ShapeDtypeStruct((C, Ts, H, W), frames.dtype),
        scratch_shapes=[
            pltpu.VMEM((C * Ts, H, W), frames.dtype),
            pltpu.SemaphoreType.DMA((C * Ts,)),
            pltpu.SemaphoreType.DMA,
        ],
    )(frames)

    return (slow, frames)


# speedup vs baseline: 4.5112x; 1.3137x over previous
import numpy as np
import jax
import jax.numpy as jnp
from jax.experimental import pallas as pl
from jax.experimental.pallas import tpu as pltpu

_ALPHA = 4
_CHUNK = 8


def kernel(frames):
    C, T, H, W = frames.shape
    Ts = T // _ALPHA
    idx = np.linspace(0, T - 1, Ts).astype(np.int32)
    nj = T // _CHUNK

    def body(in_ref, slow_ref, fast_ref, buf, sin, sout):
        ins = []
        n = 0
        for c in range(C):
            for j in range(nj):
                sl = pl.ds(j * _CHUNK, _CHUNK)
                cp = pltpu.make_async_copy(
                    in_ref.at[c, sl], buf.at[c, sl], sin.at[n])
                cp.start()
                ins.append((c, j, cp))
                n += 1
        outs = []
        for c, j, cp in ins:
            cp.wait()
            sl = pl.ds(j * _CHUNK, _CHUNK)
            o = pltpu.make_async_copy(buf.at[c, sl], fast_ref.at[c, sl], sout)
            o.start()
            outs.append(o)
            lo, hi = j * _CHUNK, (j + 1) * _CHUNK
            for p, g in enumerate(idx):
                if lo <= g < hi:
                    o2 = pltpu.make_async_copy(
                        buf.at[c, int(g)], slow_ref.at[c, p], sout)
                    o2.start()
                    outs.append(o2)
        for o in outs:
            o.wait()

    slow, fast = pl.pallas_call(
        body,
        in_specs=[pl.BlockSpec(memory_space=pl.ANY)],
        out_specs=[
            pl.BlockSpec(memory_space=pl.ANY),
            pl.BlockSpec(memory_space=pl.ANY),
        ],
        out_shape=[
            jax.ShapeDtypeStruct((C, Ts, H, W), frames.dtype),
            jax.ShapeDtypeStruct((C, T, H, W), frames.dtype),
        ],
        scratch_shapes=[
            pltpu.VMEM((C, T, H, W), frames.dtype),
            pltpu.SemaphoreType.DMA((C * nj,)),
            pltpu.SemaphoreType.DMA,
        ],
    )(frames)

    return (slow, fast)
